# Initial kernel scaffold; baseline (speedup 1.0000x reference)
import jax
import jax.numpy as jnp
from jax.experimental import pallas as pl


def kernel(z, codebook_weight):
    raise NotImplementedError("write your pallas kernel here")


# speedup vs baseline: 1.2478x; 1.2478x over previous
import functools

import jax
import jax.numpy as jnp
from jax import lax
from jax.experimental import pallas as pl
from jax.experimental.pallas import tpu as pltpu
from jax.experimental.pallas import tpu_sc as plsc

_N_CODES = 8192
_D = 32
_COMMIT = 0.25
_ROWS = 256


def _dist_argmin_body(z_ref, zsq_ref, c_ref, csq_ref, idx_ref, loss_ref):
    z = z_ref[...]
    c = c_ref[...]
    m = lax.dot_general(z.astype(jnp.bfloat16), c.astype(jnp.bfloat16),
                        dimension_numbers=(((1,), (1,)), ((), ())),
                        preferred_element_type=jnp.float32)
    dists = zsq_ref[...] - 2.0 * m + csq_ref[...]

    win = 4096
    acc_v = jnp.full((_ROWS, 1), jnp.inf, jnp.float32)
    acc_e = jnp.full((_ROWS, 1), jnp.inf, jnp.float32)
    acc_i = jnp.zeros((_ROWS, 1), jnp.int32)
    for w in range(_N_CODES // win):
        blk = dists[:, w * win:(w + 1) * win]
        v = jnp.min(blk, axis=1, keepdims=True)
        iota = lax.broadcasted_iota(jnp.int32, blk.shape, 1) + w * win
        i = jnp.min(jnp.where(blk == v, iota, _N_CODES), axis=1, keepdims=True)
        take = v < acc_v
        acc_i = jnp.where(take, i, acc_i)
        acc_e = jnp.where(take, v, acc_e)
        acc_v = jnp.where(take, v, acc_v).astype(jnp.bfloat16).astype(jnp.float32)
    idx_ref[...] = acc_i[:, 0]

    pid = pl.program_id(0)
    nsteps = pl.num_programs(0)
    blk_sum = jnp.sum(acc_e)
    prev = jnp.where(pid == 0, 0.0, loss_ref[0, 0])
    tot = prev + blk_sum
    scale = _COMMIT / (16384.0 * _D)
    loss_ref[0, 0] = jnp.where(pid == nsteps - 1, tot * scale, tot)


def _tc_dist_argmin(flat_z, zsq, codebook, csq):
    n = flat_z.shape[0]
    grid = n // _ROWS
    return pl.pallas_call(
        _dist_argmin_body,
        grid=(grid,),
        in_specs=[
            pl.BlockSpec((_ROWS, _D), lambda i: (i, 0)),
            pl.BlockSpec((_ROWS, 1), lambda i: (i, 0)),
            pl.BlockSpec((_N_CODES, _D), lambda i: (0, 0)),
            pl.BlockSpec((1, _N_CODES), lambda i: (0, 0)),
        ],
        out_specs=[
            pl.BlockSpec((_ROWS,), lambda i: (i,)),
            pl.BlockSpec(memory_space=pltpu.SMEM),
        ],
        out_shape=[
            jax.ShapeDtypeStruct((n,), jnp.int32),
            jax.ShapeDtypeStruct((1, 1), jnp.float32),
        ],
    )(flat_z, zsq, codebook, csq)


def _sc_gather(table, idx):
    info = plsc.get_sparse_core_info()
    nw = info.num_cores * info.num_subcores
    b = idx.shape[0]
    bpw = b // nw
    chunk = 128
    mesh = plsc.VectorSubcoreMesh(core_axis_name="c", subcore_axis_name="s")

    @functools.partial(
        pl.kernel, mesh=mesh,
        compiler_params=pltpu.CompilerParams(use_tc_tiling_on_sc=False),
        out_type=jax.ShapeDtypeStruct((b, _D), jnp.float32),
        scratch_types=[
            pltpu.VMEM((bpw,), jnp.int32),
            pltpu.VMEM((bpw, _D), jnp.float32),
            pltpu.SemaphoreType.DMA,
        ],
    )
    def k(table_hbm, idx_hbm, out_hbm, idx_v, rows_v, sem):
        wid = lax.axis_index("s") * info.num_cores + lax.axis_index("c")
        base = wid * bpw
        pltpu.sync_copy(idx_hbm.at[pl.ds(base, bpw)], idx_v)
        copies = [
            pltpu.async_copy(
                table_hbm.at[idx_v.at[pl.ds(j * chunk, chunk)]],
                rows_v.at[pl.ds(j * chunk, chunk)], sem)
            for j in range(bpw // chunk)
        ]
        for cp in copies:
            cp.wait()
        pltpu.sync_copy(rows_v, out_hbm.at[pl.ds(base, bpw)])

    return k(table, idx)


def kernel(z, codebook_weight):
    B, L, D = z.shape
    flat_z = z.reshape(-1, D)
    zsq = jnp.sum(flat_z ** 2, axis=1, keepdims=True)
    csq = jnp.sum(codebook_weight ** 2, axis=1, keepdims=True).T
    idx_flat, loss = _tc_dist_argmin(flat_z, zsq, codebook_weight, csq)
    z_q_flat = _sc_gather(codebook_weight, idx_flat)
    return (z_q_flat.reshape(B, L, D), idx_flat.reshape(B, L), loss[0, 0])


# speedup vs baseline: 1.2622x; 1.0115x over previous
import functools

import jax
import jax.numpy as jnp
from jax import lax
from jax.experimental import pallas as pl
from jax.experimental.pallas import tpu as pltpu
from jax.experimental.pallas import tpu_sc as plsc

_N_CODES = 8192
_D = 32
_COMMIT = 0.25
_ROWS = 512


def _dist_argmin_body(z_ref, zsq_ref, c_ref, csq_ref, idx_ref, loss_ref):
    m2 = lax.dot_general(z_ref[...], c_ref[...],
                         dimension_numbers=(((1,), (1,)), ((), ())),
                         preferred_element_type=jnp.float32)
    dists = zsq_ref[...] - m2 + csq_ref[...]

    win = 4096
    acc_v = jnp.full((_ROWS, 1), jnp.inf, jnp.float32)
    acc_e = jnp.full((_ROWS, 1), jnp.inf, jnp.float32)
    acc_i = jnp.zeros((_ROWS, 1), jnp.int32)
    for w in range(_N_CODES // win):
        blk = dists[:, w * win:(w + 1) * win]
        v = jnp.min(blk, axis=1, keepdims=True)
        iota = lax.broadcasted_iota(jnp.int32, blk.shape, 1) + w * win
        i = jnp.min(jnp.where(blk == v, iota, _N_CODES), axis=1, keepdims=True)
        take = v < acc_v
        acc_i = jnp.where(take, i, acc_i)
        acc_e = jnp.where(take, v, acc_e)
        acc_v = jnp.where(take, v, acc_v).astype(jnp.bfloat16).astype(jnp.float32)
    idx_ref[...] = acc_i[:, 0]

    pid = pl.program_id(0)
    nsteps = pl.num_programs(0)
    blk_sum = jnp.sum(acc_e)
    prev = jnp.where(pid == 0, 0.0, loss_ref[0, 0])
    tot = prev + blk_sum
    scale = _COMMIT / (16384.0 * _D)
    loss_ref[0, 0] = jnp.where(pid == nsteps - 1, tot * scale, tot)


def _tc_dist_argmin(two_z_bf16, zsq, codebook_bf16, csq):
    n = two_z_bf16.shape[0]
    grid = n // _ROWS
    return pl.pallas_call(
        _dist_argmin_body,
        grid=(grid,),
        in_specs=[
            pl.BlockSpec((_ROWS, _D), lambda i: (i, 0)),
            pl.BlockSpec((_ROWS, 1), lambda i: (i, 0)),
            pl.BlockSpec((_N_CODES, _D), lambda i: (0, 0)),
            pl.BlockSpec((1, _N_CODES), lambda i: (0, 0)),
        ],
        out_specs=[
            pl.BlockSpec((_ROWS,), lambda i: (i,)),
            pl.BlockSpec(memory_space=pltpu.SMEM),
        ],
        out_shape=[
            jax.ShapeDtypeStruct((n,), jnp.int32),
            jax.ShapeDtypeStruct((1, 1), jnp.float32),
        ],
    )(two_z_bf16, zsq, codebook_bf16, csq)


def _sc_gather(table, idx):
    info = plsc.get_sparse_core_info()
    nw = info.num_cores * info.num_subcores
    b = idx.shape[0]
    bpw = b // nw
    chunk = 128
    mesh = plsc.VectorSubcoreMesh(core_axis_name="c", subcore_axis_name="s")

    @functools.partial(
        pl.kernel, mesh=mesh,
        compiler_params=pltpu.CompilerParams(use_tc_tiling_on_sc=False),
        out_type=jax.ShapeDtypeStruct((b, _D), jnp.float32),
        scratch_types=[
            pltpu.VMEM((bpw,), jnp.int32),
            pltpu.VMEM((bpw, _D), jnp.float32),
            pltpu.SemaphoreType.DMA,
        ],
    )
    def k(table_hbm, idx_hbm, out_hbm, idx_v, rows_v, sem):
        wid = lax.axis_index("s") * info.num_cores + lax.axis_index("c")
        base = wid * bpw
        pltpu.sync_copy(idx_hbm.at[pl.ds(base, bpw)], idx_v)
        copies = [
            pltpu.async_copy(
                table_hbm.at[idx_v.at[pl.ds(j * chunk, chunk)]],
                rows_v.at[pl.ds(j * chunk, chunk)], sem)
            for j in range(bpw // chunk)
        ]
        for cp in copies:
            cp.wait()
        pltpu.sync_copy(rows_v, out_hbm.at[pl.ds(base, bpw)])

    return k(table, idx)


def kernel(z, codebook_weight):
    B, L, D = z.shape
    flat_z = z.reshape(-1, D)
    zsq = jnp.sum(flat_z ** 2, axis=1, keepdims=True)
    csq = jnp.sum(codebook_weight ** 2, axis=1, keepdims=True).T
    two_z = (2.0 * flat_z).astype(jnp.bfloat16)
    cb16 = codebook_weight.astype(jnp.bfloat16)
    idx_flat, loss = _tc_dist_argmin(two_z, zsq, cb16, csq)
    z_q_flat = _sc_gather(codebook_weight, idx_flat)
    return (z_q_flat.reshape(B, L, D), idx_flat.reshape(B, L), loss[0, 0])


# speedup vs baseline: 1.4153x; 1.1213x over previous
import functools

import jax
import jax.numpy as jnp
from jax import lax
from jax.experimental import pallas as pl
from jax.experimental.pallas import tpu as pltpu
from jax.experimental.pallas import tpu_sc as plsc

_N_CODES = 8192
_D = 32
_COMMIT = 0.25
_ROWS = 512


def _dist_argmin_body(z_ref, zsq_ref, c_ref, csq_ref, idx_ref, loss_ref):
    m2 = lax.dot_general(z_ref[...], c_ref[...],
                         dimension_numbers=(((1,), (1,)), ((), ())),
                         preferred_element_type=jnp.float32)
    dists = zsq_ref[...] - m2 + csq_ref[...]

    win = 4096
    lane = lax.broadcasted_iota(jnp.int32, (_ROWS, win), 1)
    key = lax.bitcast_convert_type(lane | jnp.int32(0x3F800000), jnp.float32)
    acc_v = jnp.full((_ROWS, 1), jnp.inf, jnp.float32)
    acc_e = jnp.full((_ROWS, 1), jnp.inf, jnp.float32)
    acc_i = jnp.zeros((_ROWS, 1), jnp.int32)
    for w in range(_N_CODES // win):
        blk = dists[:, w * win:(w + 1) * win]
        v = jnp.min(blk, axis=1, keepdims=True)
        kmin = jnp.min(jnp.where(blk == v, key, jnp.float32(2.0)),
                       axis=1, keepdims=True)
        i = (lax.bitcast_convert_type(kmin, jnp.int32)
             & jnp.int32(0x007FFFFF)) + w * win
        take = v < acc_v
        acc_i = jnp.where(take, i, acc_i)
        acc_e = jnp.where(take, v, acc_e)
        acc_v = jnp.where(take, v, acc_v).astype(jnp.bfloat16).astype(jnp.float32)
    idx_ref[...] = acc_i[:, 0].astype(jnp.int32)

    pid = pl.program_id(0)
    nsteps = pl.num_programs(0)
    blk_sum = jnp.sum(acc_e)
    prev = jnp.where(pid == 0, 0.0, loss_ref[0, 0])
    tot = prev + blk_sum
    scale = _COMMIT / (16384.0 * _D)
    loss_ref[0, 0] = jnp.where(pid == nsteps - 1, tot * scale, tot)


def _tc_dist_argmin(two_z_bf16, zsq, codebook_bf16, csq):
    n = two_z_bf16.shape[0]
    grid = n // _ROWS
    return pl.pallas_call(
        _dist_argmin_body,
        grid=(grid,),
        in_specs=[
            pl.BlockSpec((_ROWS, _D), lambda i: (i, 0)),
            pl.BlockSpec((_ROWS, 1), lambda i: (i, 0)),
            pl.BlockSpec((_N_CODES, _D), lambda i: (0, 0)),
            pl.BlockSpec((1, _N_CODES), lambda i: (0, 0)),
        ],
        out_specs=[
            pl.BlockSpec((_ROWS,), lambda i: (i,)),
            pl.BlockSpec(memory_space=pltpu.SMEM),
        ],
        out_shape=[
            jax.ShapeDtypeStruct((n,), jnp.int32),
            jax.ShapeDtypeStruct((1, 1), jnp.float32),
        ],
    )(two_z_bf16, zsq, codebook_bf16, csq)


def _sc_gather(table, idx):
    info = plsc.get_sparse_core_info()
    nw = info.num_cores * info.num_subcores
    b = idx.shape[0]
    bpw = b // nw
    chunk = 128
    mesh = plsc.VectorSubcoreMesh(core_axis_name="c", subcore_axis_name="s")

    @functools.partial(
        pl.kernel, mesh=mesh,
        compiler_params=pltpu.CompilerParams(use_tc_tiling_on_sc=False),
        out_type=jax.ShapeDtypeStruct((b, _D), jnp.float32),
        scratch_types=[
            pltpu.VMEM((bpw,), jnp.int32),
            pltpu.VMEM((bpw, _D), jnp.float32),
            pltpu.SemaphoreType.DMA,
        ],
    )
    def k(table_hbm, idx_hbm, out_hbm, idx_v, rows_v, sem):
        wid = lax.axis_index("s") * info.num_cores + lax.axis_index("c")
        base = wid * bpw
        pltpu.sync_copy(idx_hbm.at[pl.ds(base, bpw)], idx_v)
        copies = [
            pltpu.async_copy(
                table_hbm.at[idx_v.at[pl.ds(j * chunk, chunk)]],
                rows_v.at[pl.ds(j * chunk, chunk)], sem)
            for j in range(bpw // chunk)
        ]
        for cp in copies:
            cp.wait()
        pltpu.sync_copy(rows_v, out_hbm.at[pl.ds(base, bpw)])

    return k(table, idx)


def kernel(z, codebook_weight):
    B, L, D = z.shape
    flat_z = z.reshape(-1, D)
    zsq = jnp.sum(flat_z ** 2, axis=1, keepdims=True)
    csq = jnp.sum(codebook_weight ** 2, axis=1, keepdims=True).T
    two_z = (2.0 * flat_z).astype(jnp.bfloat16)
    cb16 = codebook_weight.astype(jnp.bfloat16)
    idx_flat, loss = _tc_dist_argmin(two_z, zsq, cb16, csq)
    z_q_flat = _sc_gather(codebook_weight, idx_flat)
    return (z_q_flat.reshape(B, L, D), idx_flat.reshape(B, L), loss[0, 0])
